# baseline (speedup 1.0000x reference)
import functools

import jax
import jax.numpy as jnp
from jax import lax
from jax.experimental import pallas as pl
from jax.experimental.pallas import tpu as pltpu
from jax.experimental.pallas import tpu_sc as plsc

_NUM_CORES = 2
_NUM_SUBCORES = 16
_NUM_WORKERS = _NUM_CORES * _NUM_SUBCORES
_NBUF = 2
_LANES = 16
_BBLK = 128
_HBLK = 5


def _gather_kernel(batch, hist, embed_dim, idx_hbm, table_hbm, out_hbm,
                   idxt_v0, idxt_v1, rows_v0, rows_v1, t_v0, t_v1,
                   isem, gsem, wsem):
  wid = lax.axis_index("s") * _NUM_CORES + lax.axis_index("c")
  b_per_w = batch // _NUM_WORKERS
  nbb = b_per_w // _BBLK
  nhg = hist // _HBLK
  n_chunks = nbb * nhg
  bb0 = wid * nbb
  idxt_v = (idxt_v0, idxt_v1)
  rows_v = (rows_v0, rows_v1)
  t_v = (t_v0, t_v1)

  lane = lax.iota(jnp.int32, _LANES)

  def coords(i):
    return bb0 + i % nbb, (i // nbb) * _HBLK

  def start_idx(i, s):
    bb, h0 = coords(i)
    for hj in range(_HBLK):
      pltpu.async_copy(
          idx_hbm.at[h0 + hj, pl.ds(bb * _BBLK, _BBLK)],
          idxt_v[s].at[pl.ds(hj * _BBLK, _BBLK)], isem.at[s])

  def wait_idx(i, s):
    bb, h0 = coords(i)
    for hj in range(_HBLK):
      pltpu.make_async_copy(
          idx_hbm.at[h0 + hj, pl.ds(bb * _BBLK, _BBLK)],
          idxt_v[s].at[pl.ds(hj * _BBLK, _BBLK)], isem.at[s]).wait()

  def start_gather(i, s):
    pltpu.async_copy(table_hbm.at[idxt_v[s]], rows_v[s], gsem.at[s])

  def wait_gather(i, s):
    pltpu.make_async_copy(table_hbm.at[idxt_v[s]], rows_v[s],
                          gsem.at[s]).wait()

  def transpose(s):
    src = rows_v[s]
    dst = t_v[s]
    row_consts = [lane + hj * embed_dim + k * _LANES
                  for hj in range(_HBLK) for k in range(embed_dim // _LANES)]

    def bi_body(bi, carry):
      col_idx = jnp.full((_LANES,), bi, jnp.int32)
      n = 0
      for hj in range(_HBLK):
        for k in range(embed_dim // _LANES):
          v = src[hj * _BBLK + bi, pl.ds(k * _LANES, _LANES)]
          plsc.store_scatter(dst, [row_consts[n], col_idx], v)
          n += 1
      return carry

    lax.fori_loop(0, _BBLK, bi_body, 0)

  def start_wb(i, s):
    bb, h0 = coords(i)
    for hj in range(_HBLK):
      for eb in range(embed_dim // 8):
        pltpu.async_copy(
            t_v[s].at[pl.ds(hj * embed_dim + eb * 8, 8), pl.ds(0, _BBLK)],
            out_hbm.at[h0 + hj, eb, bb], wsem.at[s])

  def wait_wb(i, s):
    bb, h0 = coords(i)
    for hj in range(_HBLK):
      for eb in range(embed_dim // 8):
        pltpu.make_async_copy(
            t_v[s].at[pl.ds(hj * embed_dim + eb * 8, 8), pl.ds(0, _BBLK)],
            out_hbm.at[h0 + hj, eb, bb], wsem.at[s]).wait()

  n_groups = n_chunks // _NBUF

  start_idx(0, 0)
  start_idx(1, 1)
  wait_idx(0, 0)
  start_gather(0, 0)
  for i in (0, 1):
    s = i % _NBUF
    wait_gather(i, s)
    start_idx(i + 2, s)
    wait_idx(i + 1, 1 - s)
    start_gather(i + 1, 1 - s)
    transpose(s)
    start_wb(i, s)

  def group_body(g, carry):
    for s in range(_NBUF):
      i = g * _NBUF + s
      wait_gather(i, s)
      start_idx(i + 2, s)
      wait_idx(i + 1, 1 - s)
      start_gather(i + 1, 1 - s)
      wait_wb(i - 2, s)
      transpose(s)
      start_wb(i, s)
    return carry

  lax.fori_loop(1, n_groups - 1, group_body, 0)

  i = n_chunks - 2
  s = i % _NBUF
  wait_gather(i, s)
  wait_idx(i + 1, 1 - s)
  start_gather(i + 1, 1 - s)
  wait_wb(i - 2, s)
  transpose(s)
  start_wb(i, s)
  i = n_chunks - 1
  s = i % _NBUF
  wait_gather(i, s)
  wait_wb(i - 2, s)
  transpose(s)
  start_wb(i, s)
  wait_wb(n_chunks - 2, (n_chunks - 2) % _NBUF)
  wait_wb(n_chunks - 1, (n_chunks - 1) % _NBUF)


_TCH = 400


def _table_transpose_kernel(n_rows, embed_dim, tt_hbm, out_hbm,
                            in0, in1, ou0, ou1, isem, osem):
  wid = lax.axis_index("s") * _NUM_CORES + lax.axis_index("c")
  n_chunks = n_rows // _TCH
  n_uni = (n_chunks // _NUM_WORKERS) * _NUM_WORKERS
  jmax = n_uni // _NUM_WORKERS
  inb = (in0, in1)
  oub = (ou0, ou1)
  lane = lax.iota(jnp.int32, _LANES)
  half = embed_dim // _LANES

  def k_of(j):
    return j * _NUM_WORKERS + wid

  def start_in(j, s):
    r0 = k_of(j) * _TCH
    pltpu.async_copy(tt_hbm.at[:, pl.ds(r0, _TCH)],
                     inb[s].at[:, pl.ds(0, _TCH)], isem.at[s])

  def wait_in(j, s):
    r0 = k_of(j) * _TCH
    pltpu.make_async_copy(tt_hbm.at[:, pl.ds(r0, _TCH)],
                          inb[s].at[:, pl.ds(0, _TCH)], isem.at[s]).wait()

  def start_out(j, s):
    r0 = k_of(j) * _TCH
    pltpu.async_copy(oub[s], out_hbm.at[pl.ds(r0, _TCH)], osem.at[s])

  def wait_out(j, s):
    r0 = k_of(j) * _TCH
    pltpu.make_async_copy(oub[s], out_hbm.at[pl.ds(r0, _TCH)],
                          osem.at[s]).wait()

  def transpose(s):
    src = inb[s]
    dst = oub[s]

    def r_body(r, carry):
      rs = jnp.full((_LANES,), r, jnp.int32)
      for k in range(half):
        v = plsc.load_gather(src, [lane + k * _LANES, rs])
        dst[r, pl.ds(k * _LANES, _LANES)] = v
      return carry

    lax.fori_loop(0, _TCH, r_body, 0)

  start_in(0, 0)
  start_in(1, 1)

  def group_body(g, carry):
    for s in range(_NBUF):
      j = g * _NBUF + s
      wait_in(j, s)

      @pl.when(j >= _NBUF)
      def _():
        wait_out(j - _NBUF, s)

      transpose(s)
      start_out(j, s)

      @pl.when(j + _NBUF < jmax)
      def _():
        start_in(j + _NBUF, s)

    return carry

  lax.fori_loop(0, jmax // _NBUF, group_body, 0)
  wait_out(jmax - 2, 0)
  wait_out(jmax - 1, 1)

  n_tail = n_chunks - n_uni

  @pl.when(wid < n_tail)
  def _():
    r0 = (n_uni + wid) * _TCH
    pltpu.sync_copy(tt_hbm.at[:, pl.ds(r0, _TCH)],
                    inb[0].at[:, pl.ds(0, _TCH)])
    transpose(0)
    pltpu.sync_copy(oub[0], out_hbm.at[pl.ds(r0, _TCH)])


def kernel(indices, table):
  batch, hist = indices.shape
  embed_dim = table.shape[1]
  eb_n = embed_dim // 8
  bb_n = batch // _BBLK
  n_tab = (table.shape[0] - 1) // _TCH * _TCH

  mesh = plsc.VectorSubcoreMesh(core_axis_name="c", subcore_axis_name="s")
  k0 = pl.kernel(
      functools.partial(_table_transpose_kernel, n_tab, embed_dim),
      out_type=jax.ShapeDtypeStruct((n_tab, embed_dim), jnp.float32),
      mesh=mesh,
      scratch_types=[
          pltpu.VMEM((embed_dim, _TCH + 1), jnp.float32),
          pltpu.VMEM((embed_dim, _TCH + 1), jnp.float32),
          pltpu.VMEM((_TCH, embed_dim), jnp.float32),
          pltpu.VMEM((_TCH, embed_dim), jnp.float32),
          pltpu.SemaphoreType.DMA((_NBUF,)),
          pltpu.SemaphoreType.DMA((_NBUF,)),
      ],
      compiler_params=pltpu.CompilerParams(use_tc_tiling_on_sc=False,
                                           needs_layout_passes=False),
  )
  k = pl.kernel(
      functools.partial(_gather_kernel, batch, hist, embed_dim),
      out_type=jax.ShapeDtypeStruct((hist, eb_n, bb_n, 8, _BBLK),
                                    jnp.float32),
      mesh=mesh,
      scratch_types=[
          pltpu.VMEM((_HBLK * _BBLK,), jnp.int32),
          pltpu.VMEM((_HBLK * _BBLK,), jnp.int32),
          pltpu.VMEM((_BBLK * _HBLK, embed_dim), jnp.float32),
          pltpu.VMEM((_BBLK * _HBLK, embed_dim), jnp.float32),
          pltpu.VMEM((_HBLK * embed_dim, _BBLK + 1), jnp.float32),
          pltpu.VMEM((_HBLK * embed_dim, _BBLK + 1), jnp.float32),
          pltpu.SemaphoreType.DMA((_NBUF,)),
          pltpu.SemaphoreType.DMA((_NBUF,)),
          pltpu.SemaphoreType.DMA((_NBUF,)),
      ],
      compiler_params=pltpu.CompilerParams(use_tc_tiling_on_sc=False,
                                           needs_layout_passes=False),
  )
  table_rm = k0(table.T)
  out5 = k(indices.T.astype(jnp.int32), table_rm)
  return out5.transpose(2, 4, 0, 1, 3).reshape(batch, hist, embed_dim)


# speedup vs baseline: 4.4323x; 4.4323x over previous
import functools

import jax
import jax.numpy as jnp
from jax import lax
from jax.experimental import pallas as pl
from jax.experimental.pallas import tpu as pltpu
from jax.experimental.pallas import tpu_sc as plsc

_NUM_CORES = 2
_NUM_SUBCORES = 16
_NUM_WORKERS = _NUM_CORES * _NUM_SUBCORES
_NBUF = 2
_LANES = 16
_BBLK = 128
_HBLK = 5


def _gather_kernel(batch, hist, embed_dim, idx_hbm, table_hbm, out_hbm,
                   idxt_v0, idxt_v1, rows_v0, rows_v1, t_v0, t_v1,
                   isem, gsem, wsem):
  wid = lax.axis_index("s") * _NUM_CORES + lax.axis_index("c")
  b_per_w = batch // _NUM_WORKERS
  nbb = b_per_w // _BBLK
  nhg = hist // _HBLK
  n_chunks = nbb * nhg
  bb0 = wid * nbb
  idxt_v = (idxt_v0, idxt_v1)
  rows_v = (rows_v0, rows_v1)
  t_v = (t_v0, t_v1)

  lane = lax.iota(jnp.int32, _LANES)

  def coords(i):
    return bb0 + i % nbb, (i // nbb) * _HBLK

  def start_idx(i, s):
    bb, h0 = coords(i)
    for hj in range(_HBLK):
      pltpu.async_copy(
          idx_hbm.at[h0 + hj, pl.ds(bb * _BBLK, _BBLK)],
          idxt_v[s].at[pl.ds(hj * _BBLK, _BBLK)], isem.at[s])

  def wait_idx(i, s):
    bb, h0 = coords(i)
    for hj in range(_HBLK):
      pltpu.make_async_copy(
          idx_hbm.at[h0 + hj, pl.ds(bb * _BBLK, _BBLK)],
          idxt_v[s].at[pl.ds(hj * _BBLK, _BBLK)], isem.at[s]).wait()

  def start_gather(i, s):
    pltpu.async_copy(table_hbm.at[idxt_v[s]], rows_v[s], gsem.at[s])

  def wait_gather(i, s):
    pltpu.make_async_copy(table_hbm.at[idxt_v[s]], rows_v[s],
                          gsem.at[s]).wait()

  def transpose(s):
    src = rows_v[s]
    dst = t_v[s]
    row_consts = [lane + hj * embed_dim + k * _LANES
                  for hj in range(_HBLK) for k in range(embed_dim // _LANES)]

    def bi_body(bi, carry):
      col_idx = jnp.full((_LANES,), bi, jnp.int32)
      n = 0
      for hj in range(_HBLK):
        for k in range(embed_dim // _LANES):
          v = src[hj * _BBLK + bi, pl.ds(k * _LANES, _LANES)]
          plsc.store_scatter(dst, [row_consts[n], col_idx], v)
          n += 1
      return carry

    lax.fori_loop(0, _BBLK, bi_body, 0)

  def start_wb(i, s):
    bb, h0 = coords(i)
    for hj in range(_HBLK):
      for eb in range(embed_dim // 8):
        pltpu.async_copy(
            t_v[s].at[pl.ds(hj * embed_dim + eb * 8, 8), pl.ds(0, _BBLK)],
            out_hbm.at[h0 + hj, eb, bb], wsem.at[s])

  def wait_wb(i, s):
    bb, h0 = coords(i)
    for hj in range(_HBLK):
      for eb in range(embed_dim // 8):
        pltpu.make_async_copy(
            t_v[s].at[pl.ds(hj * embed_dim + eb * 8, 8), pl.ds(0, _BBLK)],
            out_hbm.at[h0 + hj, eb, bb], wsem.at[s]).wait()

  n_groups = n_chunks // _NBUF

  start_idx(0, 0)
  start_idx(1, 1)
  wait_idx(0, 0)
  start_gather(0, 0)
  for i in (0, 1):
    s = i % _NBUF
    wait_gather(i, s)
    start_idx(i + 2, s)
    wait_idx(i + 1, 1 - s)
    start_gather(i + 1, 1 - s)
    transpose(s)
    start_wb(i, s)

  def group_body(g, carry):
    for s in range(_NBUF):
      i = g * _NBUF + s
      wait_gather(i, s)
      start_idx(i + 2, s)
      wait_idx(i + 1, 1 - s)
      start_gather(i + 1, 1 - s)
      wait_wb(i - 2, s)
      transpose(s)
      start_wb(i, s)
    return carry

  lax.fori_loop(1, n_groups - 1, group_body, 0)

  i = n_chunks - 2
  s = i % _NBUF
  wait_gather(i, s)
  wait_idx(i + 1, 1 - s)
  start_gather(i + 1, 1 - s)
  wait_wb(i - 2, s)
  transpose(s)
  start_wb(i, s)
  i = n_chunks - 1
  s = i % _NBUF
  wait_gather(i, s)
  wait_wb(i - 2, s)
  transpose(s)
  start_wb(i, s)
  wait_wb(n_chunks - 2, (n_chunks - 2) % _NBUF)
  wait_wb(n_chunks - 1, (n_chunks - 1) % _NBUF)


def kernel(indices, table):
  batch, hist = indices.shape
  embed_dim = table.shape[1]
  eb_n = embed_dim // 8
  bb_n = batch // _BBLK

  mesh = plsc.VectorSubcoreMesh(core_axis_name="c", subcore_axis_name="s")
  k = pl.kernel(
      functools.partial(_gather_kernel, batch, hist, embed_dim),
      out_type=jax.ShapeDtypeStruct((hist, eb_n, bb_n, 8, _BBLK),
                                    jnp.float32),
      mesh=mesh,
      scratch_types=[
          pltpu.VMEM((_HBLK * _BBLK,), jnp.int32),
          pltpu.VMEM((_HBLK * _BBLK,), jnp.int32),
          pltpu.VMEM((_BBLK * _HBLK, embed_dim), jnp.float32),
          pltpu.VMEM((_BBLK * _HBLK, embed_dim), jnp.float32),
          pltpu.VMEM((_HBLK * embed_dim, _BBLK + 1), jnp.float32),
          pltpu.VMEM((_HBLK * embed_dim, _BBLK + 1), jnp.float32),
          pltpu.SemaphoreType.DMA((_NBUF,)),
          pltpu.SemaphoreType.DMA((_NBUF,)),
          pltpu.SemaphoreType.DMA((_NBUF,)),
      ],
      compiler_params=pltpu.CompilerParams(use_tc_tiling_on_sc=False,
                                           needs_layout_passes=False),
  )
  out5 = k(indices.T.astype(jnp.int32), table)
  return out5.transpose(2, 4, 0, 1, 3).reshape(batch, hist, embed_dim)
